# baseline (speedup 1.0000x reference)
import functools

import jax
import jax.numpy as jnp
from jax import lax
from jax.experimental import pallas as pl
from jax.experimental.pallas import tpu as pltpu
from jax.experimental.pallas import tpu_sc as plsc

N_NODES = 100000
EMB_DIM = 128
NUM_EMB = 119
NBUF = 8
WINDOW = 80
NUM_BLOCKS = N_NODES // WINDOW
NUM_WORKERS = 32
BLOCKS_PER_WORKER = -(-NUM_BLOCKS // NUM_WORKERS)


def kernel(x, table):
    idx = x.reshape(N_NODES).astype(jnp.int32)
    mesh = plsc.VectorSubcoreMesh(core_axis_name="c", subcore_axis_name="s")

    @functools.partial(
        pl.kernel,
        out_type=jax.ShapeDtypeStruct((N_NODES, EMB_DIM), jnp.float32),
        mesh=mesh,
        scratch_types=(
            [pltpu.VMEM_SHARED((NUM_EMB, EMB_DIM), jnp.float32)]
            + [pltpu.VMEM((WINDOW,), jnp.int32) for _ in range(NBUF)]
            + [pltpu.VMEM((WINDOW, EMB_DIM), jnp.float32) for _ in range(NBUF)]
            + [
                pltpu.SemaphoreType.DMA((NBUF,)),
                pltpu.SemaphoreType.DMA((NBUF,)),
                pltpu.SemaphoreType.DMA((NBUF,)),
            ]
        ),
    )
    def gather_kernel(table_hbm, idx_hbm, out_hbm, table_sp, *rest):
        idx_bufs = rest[:NBUF]
        row_bufs = rest[NBUF:2 * NBUF]
        isem, gsem, wsem = rest[2 * NBUF:]
        wid = lax.axis_index("s") * 2 + lax.axis_index("c")
        nb = BLOCKS_PER_WORKER

        @pl.when(lax.axis_index("s") == 0)
        def _():
            pltpu.sync_copy(table_hbm, table_sp)

        plsc.subcore_barrier()

        def base(j):
            return (wid + j * NUM_WORKERS) * WINDOW

        def idx_copy(j):
            k = j % NBUF
            return pltpu.make_async_copy(
                idx_hbm.at[pl.ds(base(j), WINDOW)], idx_bufs[k], isem.at[k])

        def gather_copy(j):
            k = j % NBUF
            return pltpu.make_async_copy(
                table_sp.at[idx_bufs[k]], row_bufs[k], gsem.at[k])

        def write_copy(j):
            k = j % NBUF
            return pltpu.make_async_copy(
                row_bufs[k], out_hbm.at[pl.ds(base(j), WINDOW)], wsem.at[k])

        def guarded(j, fn):
            if j == nb - 1:
                @pl.when(base(j) < N_NODES)
                def _():
                    fn()
            else:
                fn()

        guarded(0, lambda: idx_copy(0).start())
        guarded(1, lambda: idx_copy(1).start())
        guarded(0, lambda: idx_copy(0).wait())
        guarded(0, lambda: gather_copy(0).start())
        for j in range(nb):
            if j + 2 < nb:
                guarded(j + 2, lambda: idx_copy(j + 2).start())
            if j + 1 < nb:
                guarded(j + 1, lambda: idx_copy(j + 1).wait())
                if j >= NBUF - 1:
                    guarded(j + 1 - NBUF, lambda: write_copy(j + 1 - NBUF).wait())
                guarded(j + 1, lambda: gather_copy(j + 1).start())
            guarded(j, lambda: gather_copy(j).wait())
            guarded(j, lambda: write_copy(j).start())
        for j in range(max(0, nb - NBUF), nb):
            guarded(j, lambda: write_copy(j).wait())

    return gather_kernel(table, idx)


# speedup vs baseline: 1.0809x; 1.0809x over previous
import functools

import jax
import jax.numpy as jnp
from jax import lax
from jax.experimental import pallas as pl
from jax.experimental.pallas import tpu as pltpu
from jax.experimental.pallas import tpu_sc as plsc

N_NODES = 100000
EMB_DIM = 128
NUM_EMB = 119
NBUF = 5
WINDOW = 200
NUM_BLOCKS = N_NODES // WINDOW
NUM_WORKERS = 32
BLOCKS_PER_WORKER = -(-NUM_BLOCKS // NUM_WORKERS)


def kernel(x, table):
    idx = x.reshape(N_NODES).astype(jnp.int32)
    mesh = plsc.VectorSubcoreMesh(core_axis_name="c", subcore_axis_name="s")

    @functools.partial(
        pl.kernel,
        out_type=jax.ShapeDtypeStruct((N_NODES, EMB_DIM), jnp.float32),
        mesh=mesh,
        scratch_types=(
            [pltpu.VMEM_SHARED((NUM_EMB, EMB_DIM), jnp.float32)]
            + [pltpu.VMEM((WINDOW,), jnp.int32) for _ in range(NBUF)]
            + [pltpu.VMEM((WINDOW, EMB_DIM), jnp.float32) for _ in range(NBUF)]
            + [
                pltpu.SemaphoreType.DMA((NBUF,)),
                pltpu.SemaphoreType.DMA((NBUF,)),
                pltpu.SemaphoreType.DMA((NBUF,)),
            ]
        ),
    )
    def gather_kernel(table_hbm, idx_hbm, out_hbm, table_sp, *rest):
        idx_bufs = rest[:NBUF]
        row_bufs = rest[NBUF:2 * NBUF]
        isem, gsem, wsem = rest[2 * NBUF:]
        wid = lax.axis_index("s") * 2 + lax.axis_index("c")
        nb = BLOCKS_PER_WORKER

        @pl.when(lax.axis_index("s") == 0)
        def _():
            pltpu.sync_copy(table_hbm, table_sp)

        plsc.subcore_barrier()

        def base(j):
            return (wid + j * NUM_WORKERS) * WINDOW

        def idx_copy(j):
            k = j % NBUF
            return pltpu.make_async_copy(
                idx_hbm.at[pl.ds(base(j), WINDOW)], idx_bufs[k], isem.at[k])

        def gather_copy(j):
            k = j % NBUF
            return pltpu.make_async_copy(
                table_sp.at[idx_bufs[k]], row_bufs[k], gsem.at[k])

        def write_copy(j):
            k = j % NBUF
            return pltpu.make_async_copy(
                row_bufs[k], out_hbm.at[pl.ds(base(j), WINDOW)], wsem.at[k])

        def guarded(j, fn):
            if j == nb - 1:
                @pl.when(base(j) < N_NODES)
                def _():
                    fn()
            else:
                fn()

        guarded(0, lambda: idx_copy(0).start())
        guarded(1, lambda: idx_copy(1).start())
        guarded(0, lambda: idx_copy(0).wait())
        guarded(0, lambda: gather_copy(0).start())
        for j in range(nb):
            if j + 2 < nb:
                guarded(j + 2, lambda: idx_copy(j + 2).start())
            if j + 1 < nb:
                guarded(j + 1, lambda: idx_copy(j + 1).wait())
                if j >= NBUF - 1:
                    guarded(j + 1 - NBUF, lambda: write_copy(j + 1 - NBUF).wait())
                guarded(j + 1, lambda: gather_copy(j + 1).start())
            guarded(j, lambda: gather_copy(j).wait())
            guarded(j, lambda: write_copy(j).start())
        for j in range(max(0, nb - NBUF), nb):
            guarded(j, lambda: write_copy(j).wait())

    return gather_kernel(table, idx)


# speedup vs baseline: 1.1099x; 1.0268x over previous
import functools

import jax
import jax.numpy as jnp
from jax import lax
from jax.experimental import pallas as pl
from jax.experimental.pallas import tpu as pltpu
from jax.experimental.pallas import tpu_sc as plsc

N_NODES = 100000
EMB_DIM = 128
NUM_EMB = 119
NBUF = 5
WINDOW = 200
NUM_BLOCKS = N_NODES // WINDOW
NUM_WORKERS = 32
BLOCKS_PER_WORKER = -(-NUM_BLOCKS // NUM_WORKERS)
STAGE_ROWS = 8


def kernel(x, table):
    idx = x.reshape(N_NODES).astype(jnp.int32)
    mesh = plsc.VectorSubcoreMesh(core_axis_name="c", subcore_axis_name="s")

    @functools.partial(
        pl.kernel,
        out_type=jax.ShapeDtypeStruct((N_NODES, EMB_DIM), jnp.float32),
        mesh=mesh,
        scratch_types=(
            [pltpu.VMEM_SHARED((NUM_EMB, EMB_DIM), jnp.float32)]
            + [pltpu.VMEM((WINDOW,), jnp.int32) for _ in range(NBUF)]
            + [pltpu.VMEM((WINDOW, EMB_DIM), jnp.float32) for _ in range(NBUF)]
            + [
                pltpu.SemaphoreType.DMA((NBUF,)),
                pltpu.SemaphoreType.DMA((NBUF,)),
                pltpu.SemaphoreType.DMA((NBUF,)),
            ]
        ),
    )
    def gather_kernel(table_hbm, idx_hbm, out_hbm, table_sp, *rest):
        idx_bufs = rest[:NBUF]
        row_bufs = rest[NBUF:2 * NBUF]
        isem, gsem, wsem = rest[2 * NBUF:]
        sid = lax.axis_index("s")
        wid = sid * 2 + lax.axis_index("c")
        nb = BLOCKS_PER_WORKER

        def base(j):
            return (wid + j * NUM_WORKERS) * WINDOW

        def idx_copy(j):
            k = j % NBUF
            return pltpu.make_async_copy(
                idx_hbm.at[pl.ds(base(j), WINDOW)], idx_bufs[k], isem.at[k])

        def gather_copy(j):
            k = j % NBUF
            return pltpu.make_async_copy(
                table_sp.at[idx_bufs[k]], row_bufs[k], gsem.at[k])

        def write_copy(j):
            k = j % NBUF
            return pltpu.make_async_copy(
                row_bufs[k], out_hbm.at[pl.ds(base(j), WINDOW)], wsem.at[k])

        def guarded(j, fn):
            if j == nb - 1:
                @pl.when(base(j) < N_NODES)
                def _():
                    fn()
            else:
                fn()

        guarded(0, lambda: idx_copy(0).start())
        guarded(1, lambda: idx_copy(1).start())

        @pl.when(sid < 14)
        def _():
            pltpu.sync_copy(table_hbm.at[pl.ds(sid * STAGE_ROWS, STAGE_ROWS)],
                            table_sp.at[pl.ds(sid * STAGE_ROWS, STAGE_ROWS)])

        @pl.when(sid == 14)
        def _():
            pltpu.sync_copy(
                table_hbm.at[pl.ds(14 * STAGE_ROWS, NUM_EMB - 14 * STAGE_ROWS)],
                table_sp.at[pl.ds(14 * STAGE_ROWS, NUM_EMB - 14 * STAGE_ROWS)])

        plsc.subcore_barrier()

        guarded(0, lambda: idx_copy(0).wait())
        guarded(0, lambda: gather_copy(0).start())
        for j in range(nb):
            if j + 2 < nb:
                guarded(j + 2, lambda: idx_copy(j + 2).start())
            if j + 1 < nb:
                guarded(j + 1, lambda: idx_copy(j + 1).wait())
                if j >= NBUF - 1:
                    guarded(j + 1 - NBUF, lambda: write_copy(j + 1 - NBUF).wait())
                guarded(j + 1, lambda: gather_copy(j + 1).start())
            guarded(j, lambda: gather_copy(j).wait())
            guarded(j, lambda: write_copy(j).start())
        for j in range(max(0, nb - NBUF), nb):
            guarded(j, lambda: write_copy(j).wait())

    return gather_kernel(table, idx)


# speedup vs baseline: 1.1314x; 1.0194x over previous
import functools

import jax
import jax.numpy as jnp
from jax import lax
from jax.experimental import pallas as pl
from jax.experimental.pallas import tpu as pltpu
from jax.experimental.pallas import tpu_sc as plsc

N_NODES = 100000
EMB_DIM = 128
NUM_EMB = 119
NBUF = 3
WINDOW = 200
NUM_MAIN = 15
SMALL_A = 128
SMALL_B = 120
NUM_WORKERS = 32
NB = NUM_MAIN + 1
STAGE_ROWS = 8


def kernel(x, table):
    idx = x.reshape(N_NODES).astype(jnp.int32)
    mesh = plsc.VectorSubcoreMesh(core_axis_name="c", subcore_axis_name="s")

    @functools.partial(
        pl.kernel,
        out_type=jax.ShapeDtypeStruct((N_NODES, EMB_DIM), jnp.float32),
        mesh=mesh,
        scratch_types=(
            [pltpu.VMEM_SHARED((NUM_EMB, EMB_DIM), jnp.float32)]
            + [pltpu.VMEM((WINDOW,), jnp.int32) for _ in range(NBUF)]
            + [pltpu.VMEM((WINDOW, EMB_DIM), jnp.float32) for _ in range(NBUF)]
            + [
                pltpu.VMEM((SMALL_A,), jnp.int32),
                pltpu.VMEM((SMALL_A, EMB_DIM), jnp.float32),
                pltpu.VMEM((SMALL_B,), jnp.int32),
                pltpu.VMEM((SMALL_B, EMB_DIM), jnp.float32),
                pltpu.SemaphoreType.DMA((NBUF + 1,)),
                pltpu.SemaphoreType.DMA((NBUF + 1,)),
                pltpu.SemaphoreType.DMA((NBUF + 1,)),
            ]
        ),
    )
    def gather_kernel(table_hbm, idx_hbm, out_hbm, table_sp, *rest):
        idx_bufs = rest[:NBUF]
        row_bufs = rest[NBUF:2 * NBUF]
        t_idx_a, t_rows_a, t_idx_b, t_rows_b = rest[2 * NBUF:2 * NBUF + 4]
        isem, gsem, wsem = rest[2 * NBUF + 4:]
        sid = lax.axis_index("s")
        wid = sid * 2 + lax.axis_index("c")

        start_w = wid * 3120 + 8 * jnp.minimum(wid, 20)
        tsize = jnp.where(wid < 20, SMALL_A, SMALL_B)
        is_a = wid < 20

        def mbase(j):
            return start_w + tsize + WINDOW * (j - 1)

        def midx(j):
            k = (j - 1) % NBUF
            return pltpu.make_async_copy(
                idx_hbm.at[pl.ds(mbase(j), WINDOW)], idx_bufs[k], isem.at[k])

        def mgather(j):
            k = (j - 1) % NBUF
            return pltpu.make_async_copy(
                table_sp.at[idx_bufs[k]], row_bufs[k], gsem.at[k])

        def mwrite(j):
            k = (j - 1) % NBUF
            return pltpu.make_async_copy(
                row_bufs[k], out_hbm.at[pl.ds(mbase(j), WINDOW)], wsem.at[k])

        def small_op(fn):
            @pl.when(is_a)
            def _():
                fn(t_idx_a, t_rows_a, SMALL_A)

            @pl.when(jnp.logical_not(is_a))
            def _():
                fn(t_idx_b, t_rows_b, SMALL_B)

        def s_idx(ib, rb, n):
            return pltpu.make_async_copy(
                idx_hbm.at[pl.ds(start_w, n)], ib, isem.at[NBUF])

        def s_gather(ib, rb, n):
            return pltpu.make_async_copy(
                table_sp.at[ib], rb, gsem.at[NBUF])

        def s_write(ib, rb, n):
            return pltpu.make_async_copy(
                rb, out_hbm.at[pl.ds(start_w, n)], wsem.at[NBUF])

        small_op(lambda ib, rb, n: s_idx(ib, rb, n).start())
        midx(1).start()

        @pl.when(sid < 14)
        def _():
            pltpu.sync_copy(table_hbm.at[pl.ds(sid * STAGE_ROWS, STAGE_ROWS)],
                            table_sp.at[pl.ds(sid * STAGE_ROWS, STAGE_ROWS)])

        @pl.when(sid == 14)
        def _():
            pltpu.sync_copy(
                table_hbm.at[pl.ds(14 * STAGE_ROWS, NUM_EMB - 14 * STAGE_ROWS)],
                table_sp.at[pl.ds(14 * STAGE_ROWS, NUM_EMB - 14 * STAGE_ROWS)])

        plsc.subcore_barrier()

        small_op(lambda ib, rb, n: s_idx(ib, rb, n).wait())
        small_op(lambda ib, rb, n: s_gather(ib, rb, n).start())

        for j in range(NB):
            if j + 2 < NB:
                midx(j + 2).start()
            if j + 1 < NB:
                midx(j + 1).wait()
                if j + 1 - NBUF >= 1:
                    mwrite(j + 1 - NBUF).wait()
                mgather(j + 1).start()
            if j == 0:
                small_op(lambda ib, rb, n: s_gather(ib, rb, n).wait())
                small_op(lambda ib, rb, n: s_write(ib, rb, n).start())
            else:
                mgather(j).wait()
                mwrite(j).start()
        small_op(lambda ib, rb, n: s_write(ib, rb, n).wait())
        for j in range(max(1, NB - NBUF), NB):
            mwrite(j).wait()

    return gather_kernel(table, idx)


# speedup vs baseline: 1.1406x; 1.0081x over previous
import functools

import jax
import jax.numpy as jnp
from jax import lax
from jax.experimental import pallas as pl
from jax.experimental.pallas import tpu as pltpu
from jax.experimental.pallas import tpu_sc as plsc

N_NODES = 100000
EMB_DIM = 128
NUM_EMB = 119
NBUF = 3
WINDOW = 200
NUM_MAIN = 15
SMALL_A = 128
SMALL_B = 120
NUM_WORKERS = 32
NB = NUM_MAIN + 1
STAGE_ROWS = 8


def kernel(x, table):
    idx = x.reshape(N_NODES).astype(jnp.int32)
    mesh = plsc.VectorSubcoreMesh(core_axis_name="c", subcore_axis_name="s")

    @functools.partial(
        pl.kernel,
        out_type=jax.ShapeDtypeStruct((N_NODES, EMB_DIM), jnp.float32),
        mesh=mesh,
        scratch_types=(
            [pltpu.VMEM_SHARED((NUM_EMB, EMB_DIM), jnp.float32)]
            + [pltpu.VMEM((NUM_MAIN * WINDOW,), jnp.int32)]
            + [pltpu.VMEM((WINDOW, EMB_DIM), jnp.float32) for _ in range(NBUF)]
            + [
                pltpu.VMEM((SMALL_A,), jnp.int32),
                pltpu.VMEM((SMALL_A, EMB_DIM), jnp.float32),
                pltpu.VMEM((SMALL_B,), jnp.int32),
                pltpu.VMEM((SMALL_B, EMB_DIM), jnp.float32),
                pltpu.SemaphoreType.DMA((2,)),
                pltpu.SemaphoreType.DMA((NBUF + 1,)),
                pltpu.SemaphoreType.DMA((NBUF + 1,)),
            ]
        ),
    )
    def gather_kernel(table_hbm, idx_hbm, out_hbm, table_sp, idx_all, rv0, rv1,
                      rv2, t_idx_a, t_rows_a, t_idx_b, t_rows_b, isem, gsem,
                      wsem):
        row_bufs = (rv0, rv1, rv2)
        sid = lax.axis_index("s")
        wid = sid * 2 + lax.axis_index("c")

        start_w = wid * 3120 + 8 * jnp.minimum(wid, 20)
        tsize = jnp.where(wid < 20, SMALL_A, SMALL_B)
        is_a = wid < 20

        def mbase(j):
            return start_w + tsize + WINDOW * (j - 1)

        def idx_all_copy():
            return pltpu.make_async_copy(
                idx_hbm.at[pl.ds(start_w + tsize, NUM_MAIN * WINDOW)],
                idx_all, isem.at[0])

        def mgather(j):
            k = (j - 1) % NBUF
            return pltpu.make_async_copy(
                table_sp.at[idx_all.at[pl.ds(WINDOW * (j - 1), WINDOW)]],
                row_bufs[k], gsem.at[k])

        def mwrite(j):
            k = (j - 1) % NBUF
            return pltpu.make_async_copy(
                row_bufs[k], out_hbm.at[pl.ds(mbase(j), WINDOW)], wsem.at[k])

        def small_op(fn):
            @pl.when(is_a)
            def _():
                fn(t_idx_a, t_rows_a, SMALL_A)

            @pl.when(jnp.logical_not(is_a))
            def _():
                fn(t_idx_b, t_rows_b, SMALL_B)

        def s_idx(ib, rb, n):
            return pltpu.make_async_copy(
                idx_hbm.at[pl.ds(start_w, n)], ib, isem.at[1])

        def s_gather(ib, rb, n):
            return pltpu.make_async_copy(
                table_sp.at[ib], rb, gsem.at[NBUF])

        def s_write(ib, rb, n):
            return pltpu.make_async_copy(
                rb, out_hbm.at[pl.ds(start_w, n)], wsem.at[NBUF])

        small_op(lambda ib, rb, n: s_idx(ib, rb, n).start())
        idx_all_copy().start()

        @pl.when(sid < 14)
        def _():
            pltpu.sync_copy(table_hbm.at[pl.ds(sid * STAGE_ROWS, STAGE_ROWS)],
                            table_sp.at[pl.ds(sid * STAGE_ROWS, STAGE_ROWS)])

        @pl.when(sid == 14)
        def _():
            pltpu.sync_copy(
                table_hbm.at[pl.ds(14 * STAGE_ROWS, NUM_EMB - 14 * STAGE_ROWS)],
                table_sp.at[pl.ds(14 * STAGE_ROWS, NUM_EMB - 14 * STAGE_ROWS)])

        plsc.subcore_barrier()

        small_op(lambda ib, rb, n: s_idx(ib, rb, n).wait())
        small_op(lambda ib, rb, n: s_gather(ib, rb, n).start())
        idx_all_copy().wait()

        for j in range(NB):
            if j + 1 < NB:
                if j + 1 - NBUF >= 1:
                    mwrite(j + 1 - NBUF).wait()
                mgather(j + 1).start()
            if j == 0:
                small_op(lambda ib, rb, n: s_gather(ib, rb, n).wait())
                small_op(lambda ib, rb, n: s_write(ib, rb, n).start())
            else:
                mgather(j).wait()
                mwrite(j).start()
        small_op(lambda ib, rb, n: s_write(ib, rb, n).wait())
        for j in range(max(1, NB - NBUF), NB):
            mwrite(j).wait()

    return gather_kernel(table, idx)
